# baseline (speedup 1.0000x reference)
import functools

import jax
import jax.numpy as jnp
from jax import lax
from jax.experimental import pallas as pl
from jax.experimental.pallas import tpu as pltpu
from jax.experimental.pallas import tpu_sc as plsc


def kernel(x_user, x_track_pos, x_track_neg, user_mat, track_mat):
    B = x_user.shape[0]
    D = user_mat.shape[1]
    info = plsc.get_sparse_core_info()
    NW = info.num_cores * info.num_subcores
    b_per_w = B // NW

    mesh = plsc.VectorSubcoreMesh(core_axis_name="c", subcore_axis_name="s")
    out_sds = jax.ShapeDtypeStruct((B, D), jnp.float32)

    @functools.partial(
        pl.kernel,
        mesh=mesh,
        out_type=(out_sds, out_sds, out_sds),
        scratch_types=[
            pltpu.VMEM((b_per_w,), jnp.int32),
            pltpu.VMEM((b_per_w,), jnp.int32),
            pltpu.VMEM((b_per_w,), jnp.int32),
            pltpu.VMEM((b_per_w, D), jnp.float32),
            pltpu.VMEM((b_per_w, D), jnp.float32),
            pltpu.VMEM((b_per_w, D), jnp.float32),
            pltpu.SemaphoreType.DMA,
            pltpu.SemaphoreType.DMA,
            pltpu.SemaphoreType.DMA,
            pltpu.SemaphoreType.DMA,
        ],
        compiler_params=pltpu.CompilerParams(use_tc_tiling_on_sc=False),
    )
    def gather3(xu, xp, xn, um, tm, out_u, out_p, out_n,
                idx_u, idx_p, idx_n, rows_u, rows_p, rows_n,
                sem_u, sem_p, sem_n, sem_o):
        wid = lax.axis_index("s") * info.num_cores + lax.axis_index("c")
        sl = pl.ds(wid * b_per_w, b_per_w)
        pltpu.sync_copy(xu.at[sl], idx_u)
        pltpu.sync_copy(xp.at[sl], idx_p)
        pltpu.sync_copy(xn.at[sl], idx_n)
        g_u = pltpu.async_copy(um.at[idx_u], rows_u, sem_u)
        g_p = pltpu.async_copy(tm.at[idx_p], rows_p, sem_p)
        g_n = pltpu.async_copy(tm.at[idx_n], rows_n, sem_n)
        g_u.wait()
        w_u = pltpu.async_copy(rows_u, out_u.at[sl], sem_o)
        g_p.wait()
        w_p = pltpu.async_copy(rows_p, out_p.at[sl], sem_o)
        g_n.wait()
        w_n = pltpu.async_copy(rows_n, out_n.at[sl], sem_o)
        w_u.wait()
        w_p.wait()
        w_n.wait()

    return tuple(gather3(x_user, x_track_pos, x_track_neg, user_mat, track_mat))
